# baseline (speedup 1.0000x reference)
import jax
import jax.numpy as jnp
from jax.experimental import pallas as pl
from jax.experimental.pallas import tpu as pltpu

_N_BINS = 15
_LANES = 128
_BLOCK = 4096


def _body(bounds_ref, logits_ref, labels_ref, out_ref, acc_ref, *, grid, n_total, n_cols):
    i = pl.program_id(0)
    x = logits_ref[...]
    m = jnp.max(x, axis=1, keepdims=True)
    e = jnp.exp(x - m)
    s = jnp.sum(e, axis=1)
    conf = 1.0 / s

    col = jax.lax.broadcasted_iota(jnp.int32, x.shape, 1)
    pred = jnp.min(jnp.where(x == m, col, n_cols), axis=1)
    lab = labels_ref[0, 0, :]
    accv = (pred == lab).astype(jnp.float32)

    cb = conf[:, None]
    lowers = bounds_ref[0:1, :]
    uppers = bounds_ref[1:2, :]
    onehot = ((cb > lowers) & (cb <= uppers)).astype(jnp.float32)

    cnt_p = jnp.sum(onehot, axis=0, keepdims=True)
    conf_p = jnp.sum(onehot * cb, axis=0, keepdims=True)
    acc_p = jnp.sum(onehot * accv[:, None], axis=0, keepdims=True)
    zeros5 = jnp.zeros((5, _LANES), jnp.float32)
    part = jnp.concatenate([cnt_p, conf_p, acc_p, zeros5], axis=0)

    @pl.when(i == 0)
    def _():
        acc_ref[...] = part

    @pl.when(i > 0)
    def _():
        acc_ref[...] = acc_ref[...] + part

    @pl.when(i == grid - 1)
    def _():
        acc = acc_ref[...]
        cnt = acc[0:1, :]
        confs = acc[1:2, :]
        accs = acc[2:3, :]
        denom = jnp.maximum(cnt, 1.0)
        acc_in = accs / denom
        conf_in = confs / denom
        gap = jnp.abs(conf_in - acc_in)
        nonempty = cnt > 0.0
        ece = jnp.sum(jnp.where(nonempty, gap * (cnt / n_total), 0.0))
        mce = jnp.max(jnp.where(nonempty, gap, -jnp.inf))
        mce = jnp.where(jnp.isneginf(mce), jnp.float32(1.0), mce)
        lane = jax.lax.broadcasted_iota(jnp.int32, (1, _LANES), 1)
        out_ref[...] = jnp.where(lane == 0, ece, jnp.where(lane == 1, mce, 0.0))


def kernel(logits, labels):
    n, c = logits.shape
    block = min(_BLOCK, n)
    grid = n // block

    bb = jnp.linspace(0.0, 1.0, _N_BINS + 1).astype(jnp.float32)
    low_pad = jnp.full((_LANES,), 3.0, jnp.float32).at[:_N_BINS].set(bb[:-1])
    up_pad = jnp.full((_LANES,), 4.0, jnp.float32).at[:_N_BINS].set(bb[1:])
    bounds = jnp.stack([low_pad, up_pad])

    labels_r = labels.astype(jnp.int32).reshape(grid, 1, block)

    import functools
    body = functools.partial(_body, grid=grid, n_total=float(n), n_cols=c)
    out = pl.pallas_call(
        body,
        grid=(grid,),
        in_specs=[
            pl.BlockSpec((2, _LANES), lambda i: (0, 0)),
            pl.BlockSpec((block, c), lambda i: (i, 0)),
            pl.BlockSpec((1, 1, block), lambda i: (i, 0, 0)),
        ],
        out_specs=pl.BlockSpec((1, _LANES), lambda i: (0, 0)),
        out_shape=jax.ShapeDtypeStruct((1, _LANES), jnp.float32),
        scratch_shapes=[pltpu.VMEM((8, _LANES), jnp.float32)],
    )(bounds, logits, labels_r)

    ece = out[0, 0:1]
    mce = out[0, 1]
    return (ece, mce)


# speedup vs baseline: 1.0187x; 1.0187x over previous
import functools

import jax
import jax.numpy as jnp
from jax.experimental import pallas as pl
from jax.experimental.pallas import tpu as pltpu

_N_BINS = 15
_LANES = 128
_BLOCK = 4096


def _body(bounds_ref, colf_ref, logits_ref, labels_ref, out_ref, acc_ref, *, grid, n_total, n_cols):
    i = pl.program_id(0)
    x = logits_ref[...]
    m = jnp.max(x, axis=1, keepdims=True)
    e = jnp.exp(x - m)
    s = jnp.sum(e, axis=1, keepdims=True)
    conf = 1.0 / s

    colf = colf_ref[...]
    predf = jnp.min(jnp.where(x == m, colf, float(n_cols)), axis=1, keepdims=True)
    accb = predf == labels_ref[...]

    lowers = bounds_ref[0:1, :]
    uppers = bounds_ref[1:2, :]
    oh = (conf > lowers) & (conf <= uppers)
    one = jnp.float32(1.0)
    zero = jnp.float32(0.0)
    z1 = jnp.where(oh, one, zero)
    z2 = jnp.where(oh, conf, zero)
    z3 = jnp.where(oh & accb, one, zero)

    cnt_p = jnp.sum(z1, axis=0, keepdims=True)
    conf_p = jnp.sum(z2, axis=0, keepdims=True)
    acc_p = jnp.sum(z3, axis=0, keepdims=True)
    zeros5 = jnp.zeros((5, _LANES), jnp.float32)
    part = jnp.concatenate([cnt_p, conf_p, acc_p, zeros5], axis=0)

    @pl.when(i == 0)
    def _():
        acc_ref[...] = part

    @pl.when(i > 0)
    def _():
        acc_ref[...] = acc_ref[...] + part

    @pl.when(i == grid - 1)
    def _():
        acc = acc_ref[...]
        cnt = acc[0:1, :]
        confs = acc[1:2, :]
        accs = acc[2:3, :]
        denom = jnp.maximum(cnt, 1.0)
        acc_in = accs / denom
        conf_in = confs / denom
        gap = jnp.abs(conf_in - acc_in)
        nonempty = cnt > 0.0
        ece = jnp.sum(jnp.where(nonempty, gap * (cnt / n_total), 0.0))
        mce = jnp.max(jnp.where(nonempty, gap, -jnp.inf))
        mce = jnp.where(jnp.isneginf(mce), jnp.float32(1.0), mce)
        lane = jax.lax.broadcasted_iota(jnp.int32, (1, _LANES), 1)
        out_ref[...] = jnp.where(lane == 0, ece, jnp.where(lane == 1, mce, 0.0))


def kernel(logits, labels):
    n, c = logits.shape
    block = min(_BLOCK, n)
    grid = n // block

    bb = jnp.linspace(0.0, 1.0, _N_BINS + 1).astype(jnp.float32)
    low_pad = jnp.full((_LANES,), 3.0, jnp.float32).at[:_N_BINS].set(bb[:-1])
    up_pad = jnp.full((_LANES,), 4.0, jnp.float32).at[:_N_BINS].set(bb[1:])
    bounds = jnp.stack([low_pad, up_pad])

    labels_f = labels.astype(jnp.float32).reshape(n, 1)
    colf = jnp.arange(c, dtype=jnp.float32).reshape(1, c)

    body = functools.partial(_body, grid=grid, n_total=float(n), n_cols=c)
    out = pl.pallas_call(
        body,
        grid=(grid,),
        in_specs=[
            pl.BlockSpec((2, _LANES), lambda i: (0, 0)),
            pl.BlockSpec((1, c), lambda i: (0, 0)),
            pl.BlockSpec((block, c), lambda i: (i, 0)),
            pl.BlockSpec((block, 1), lambda i: (i, 0)),
        ],
        out_specs=pl.BlockSpec((1, _LANES), lambda i: (0, 0)),
        out_shape=jax.ShapeDtypeStruct((1, _LANES), jnp.float32),
        scratch_shapes=[pltpu.VMEM((8, _LANES), jnp.float32)],
    )(bounds, colf, logits, labels_f)

    ece = out[0, 0:1]
    mce = out[0, 1]
    return (ece, mce)


# speedup vs baseline: 1.0466x; 1.0274x over previous
import functools

import jax
import jax.numpy as jnp
from jax.experimental import pallas as pl
from jax.experimental.pallas import tpu as pltpu

_N_BINS = 15
_LANES = 128
_BLOCK = 8192


def _body(bounds_ref, colf_ref, logits_ref, labels_ref, out_ref, acc_ref, *, grid, n_total, n_cols):
    i = pl.program_id(0)
    x = logits_ref[...]
    m = jnp.max(x, axis=1, keepdims=True)
    e = jnp.exp(x - m)
    s = jnp.sum(e, axis=1, keepdims=True)
    conf = 1.0 / s

    colf = colf_ref[...]
    predf = jnp.min(jnp.where(x == m, colf, float(n_cols)), axis=1, keepdims=True)
    accb = predf == labels_ref[...]

    lowers = bounds_ref[0:1, :]
    uppers = bounds_ref[1:2, :]
    oh = (conf > lowers) & (conf <= uppers)
    one = jnp.float32(1.0)
    zero = jnp.float32(0.0)
    z1 = jnp.where(oh, one, zero)
    z2 = jnp.where(oh, conf, zero)
    z3 = jnp.where(oh & accb, one, zero)

    cnt_p = jnp.sum(z1, axis=0, keepdims=True)
    conf_p = jnp.sum(z2, axis=0, keepdims=True)
    acc_p = jnp.sum(z3, axis=0, keepdims=True)
    zeros5 = jnp.zeros((5, _LANES), jnp.float32)
    part = jnp.concatenate([cnt_p, conf_p, acc_p, zeros5], axis=0)

    @pl.when(i == 0)
    def _():
        acc_ref[...] = part

    @pl.when(i > 0)
    def _():
        acc_ref[...] = acc_ref[...] + part

    @pl.when(i == grid - 1)
    def _():
        acc = acc_ref[...]
        cnt = acc[0:1, :]
        confs = acc[1:2, :]
        accs = acc[2:3, :]
        denom = jnp.maximum(cnt, 1.0)
        acc_in = accs / denom
        conf_in = confs / denom
        gap = jnp.abs(conf_in - acc_in)
        nonempty = cnt > 0.0
        ece = jnp.sum(jnp.where(nonempty, gap * (cnt / n_total), 0.0))
        mce = jnp.max(jnp.where(nonempty, gap, -jnp.inf))
        mce = jnp.where(jnp.isneginf(mce), jnp.float32(1.0), mce)
        lane = jax.lax.broadcasted_iota(jnp.int32, (1, _LANES), 1)
        out_ref[...] = jnp.where(lane == 0, ece, jnp.where(lane == 1, mce, 0.0))


def kernel(logits, labels):
    n, c = logits.shape
    block = min(_BLOCK, n)
    grid = n // block

    bb = jnp.linspace(0.0, 1.0, _N_BINS + 1).astype(jnp.float32)
    low_pad = jnp.full((_LANES,), 3.0, jnp.float32).at[:_N_BINS].set(bb[:-1])
    up_pad = jnp.full((_LANES,), 4.0, jnp.float32).at[:_N_BINS].set(bb[1:])
    bounds = jnp.stack([low_pad, up_pad])

    labels_f = labels.astype(jnp.float32).reshape(n, 1)
    colf = jnp.arange(c, dtype=jnp.float32).reshape(1, c)

    body = functools.partial(_body, grid=grid, n_total=float(n), n_cols=c)
    out = pl.pallas_call(
        body,
        grid=(grid,),
        in_specs=[
            pl.BlockSpec((2, _LANES), lambda i: (0, 0)),
            pl.BlockSpec((1, c), lambda i: (0, 0)),
            pl.BlockSpec((block, c), lambda i: (i, 0)),
            pl.BlockSpec((block, 1), lambda i: (i, 0)),
        ],
        out_specs=pl.BlockSpec((1, _LANES), lambda i: (0, 0)),
        out_shape=jax.ShapeDtypeStruct((1, _LANES), jnp.float32),
        scratch_shapes=[pltpu.VMEM((8, _LANES), jnp.float32)],
    )(bounds, colf, logits, labels_f)

    ece = out[0, 0:1]
    mce = out[0, 1]
    return (ece, mce)


# speedup vs baseline: 2.1337x; 2.0387x over previous
import functools

import jax
import jax.numpy as jnp
from jax.experimental import pallas as pl
from jax.experimental.pallas import tpu as pltpu

_N_BINS = 15
_LANES = 128
_BLOCK = 8192


def _body(bounds_ref, logits_ref, labels_ref, out_ref, acc_ref, *, grid, n_total, n_cols):
    i = pl.program_id(0)
    b = logits_ref.shape[0]
    xt = jnp.transpose(logits_ref[...], (1, 0))
    m = jnp.max(xt, axis=0, keepdims=True)
    e = jnp.exp(xt - m)
    s = jnp.sum(e, axis=0, keepdims=True)
    conf = 1.0 / s

    col = jax.lax.broadcasted_iota(jnp.int32, xt.shape, 0)
    pred = jnp.min(jnp.where(xt == m, col, n_cols), axis=0, keepdims=True)
    accb = pred == labels_ref[0]

    lo = bounds_ref[:, 0:1]
    up = bounds_ref[:, 1:2]
    oh = (conf > lo) & (conf <= up)
    one = jnp.float32(1.0)
    zero = jnp.float32(0.0)
    z1 = jnp.where(oh, one, zero)
    z2 = jnp.where(oh, conf, zero)
    z3 = jnp.where(oh & accb, one, zero)

    def fold(z):
        t = z[:, 0:_LANES]
        for g in range(1, b // _LANES):
            t = t + z[:, g * _LANES:(g + 1) * _LANES]
        return t

    part = jnp.concatenate([fold(z1), fold(z2), fold(z3)], axis=0)

    @pl.when(i == 0)
    def _():
        acc_ref[...] = part

    @pl.when(i > 0)
    def _():
        acc_ref[...] = acc_ref[...] + part

    @pl.when(i == grid - 1)
    def _():
        acc = acc_ref[...]
        cnt = jnp.sum(acc[0:16, :], axis=1, keepdims=True)
        confs = jnp.sum(acc[16:32, :], axis=1, keepdims=True)
        accs = jnp.sum(acc[32:48, :], axis=1, keepdims=True)
        denom = jnp.maximum(cnt, 1.0)
        acc_in = accs / denom
        conf_in = confs / denom
        gap = jnp.abs(conf_in - acc_in)
        nonempty = cnt > 0.0
        ece = jnp.sum(jnp.where(nonempty, gap * (cnt / n_total), 0.0))
        mce = jnp.max(jnp.where(nonempty, gap, -jnp.inf))
        mce = jnp.where(jnp.isneginf(mce), jnp.float32(1.0), mce)
        lane = jax.lax.broadcasted_iota(jnp.int32, (1, _LANES), 1)
        out_ref[...] = jnp.where(lane == 0, ece, jnp.where(lane == 1, mce, 0.0))


def kernel(logits, labels):
    n, c = logits.shape
    block = min(_BLOCK, n)
    grid = n // block

    bb = jnp.linspace(0.0, 1.0, _N_BINS + 1).astype(jnp.float32)
    lo_col = jnp.full((16,), 3.0, jnp.float32).at[:_N_BINS].set(bb[:-1])
    up_col = jnp.full((16,), 4.0, jnp.float32).at[:_N_BINS].set(bb[1:])
    bounds = jnp.zeros((16, _LANES), jnp.float32)
    bounds = bounds.at[:, 0].set(lo_col).at[:, 1].set(up_col)

    labels_r = labels.astype(jnp.int32).reshape(grid, 1, block)

    body = functools.partial(_body, grid=grid, n_total=float(n), n_cols=c)
    out = pl.pallas_call(
        body,
        grid=(grid,),
        in_specs=[
            pl.BlockSpec((16, _LANES), lambda i: (0, 0)),
            pl.BlockSpec((block, c), lambda i: (i, 0)),
            pl.BlockSpec((1, 1, block), lambda i: (i, 0, 0)),
        ],
        out_specs=pl.BlockSpec((1, _LANES), lambda i: (0, 0)),
        out_shape=jax.ShapeDtypeStruct((1, _LANES), jnp.float32),
        scratch_shapes=[pltpu.VMEM((48, _LANES), jnp.float32)],
    )(bounds, logits, labels_r)

    ece = out[0, 0:1]
    mce = out[0, 1]
    return (ece, mce)


# speedup vs baseline: 2.2816x; 1.0693x over previous
import functools

import jax
import jax.numpy as jnp
from jax.experimental import pallas as pl
from jax.experimental.pallas import tpu as pltpu

_N_BINS = 15
_LANES = 128
_BLOCK = 16384


def _body(bounds_ref, logits_ref, labels_ref, out_ref, acc_ref, *, grid, n_total, n_cols):
    i = pl.program_id(0)
    b = logits_ref.shape[0]
    xt = jnp.transpose(logits_ref[...], (1, 0))
    m = jnp.max(xt, axis=0, keepdims=True)
    e = jnp.exp(xt - m)
    s = jnp.sum(e, axis=0, keepdims=True)
    conf = 1.0 / s

    col = jax.lax.broadcasted_iota(jnp.int32, xt.shape, 0)
    pred = jnp.min(jnp.where(xt == m, col, n_cols), axis=0, keepdims=True)
    accb = pred == labels_ref[0]

    lo = bounds_ref[:, 0:1]
    up = bounds_ref[:, 1:2]
    oh = (conf > lo) & (conf <= up)
    one = jnp.float32(1.0)
    zero = jnp.float32(0.0)
    z1 = jnp.where(oh, one, zero)
    z2 = jnp.where(oh, conf, zero)
    z3 = jnp.where(oh & accb, one, zero)

    def fold(z):
        t = z[:, 0:_LANES]
        for g in range(1, b // _LANES):
            t = t + z[:, g * _LANES:(g + 1) * _LANES]
        return t

    part = jnp.concatenate([fold(z1), fold(z2), fold(z3)], axis=0)

    @pl.when(i == 0)
    def _():
        acc_ref[...] = part

    @pl.when(i > 0)
    def _():
        acc_ref[...] = acc_ref[...] + part

    @pl.when(i == grid - 1)
    def _():
        acc = acc_ref[...]
        cnt = jnp.sum(acc[0:16, :], axis=1, keepdims=True)
        confs = jnp.sum(acc[16:32, :], axis=1, keepdims=True)
        accs = jnp.sum(acc[32:48, :], axis=1, keepdims=True)
        denom = jnp.maximum(cnt, 1.0)
        acc_in = accs / denom
        conf_in = confs / denom
        gap = jnp.abs(conf_in - acc_in)
        nonempty = cnt > 0.0
        ece = jnp.sum(jnp.where(nonempty, gap * (cnt / n_total), 0.0))
        mce = jnp.max(jnp.where(nonempty, gap, -jnp.inf))
        mce = jnp.where(jnp.isneginf(mce), jnp.float32(1.0), mce)
        lane = jax.lax.broadcasted_iota(jnp.int32, (1, _LANES), 1)
        out_ref[...] = jnp.where(lane == 0, ece, jnp.where(lane == 1, mce, 0.0))


def kernel(logits, labels):
    n, c = logits.shape
    block = min(_BLOCK, n)
    grid = n // block

    bb = jnp.linspace(0.0, 1.0, _N_BINS + 1).astype(jnp.float32)
    lo_col = jnp.full((16,), 3.0, jnp.float32).at[:_N_BINS].set(bb[:-1])
    up_col = jnp.full((16,), 4.0, jnp.float32).at[:_N_BINS].set(bb[1:])
    bounds = jnp.zeros((16, _LANES), jnp.float32)
    bounds = bounds.at[:, 0].set(lo_col).at[:, 1].set(up_col)

    labels_r = labels.astype(jnp.int32).reshape(grid, 1, block)

    body = functools.partial(_body, grid=grid, n_total=float(n), n_cols=c)
    out = pl.pallas_call(
        body,
        grid=(grid,),
        in_specs=[
            pl.BlockSpec((16, _LANES), lambda i: (0, 0)),
            pl.BlockSpec((block, c), lambda i: (i, 0)),
            pl.BlockSpec((1, 1, block), lambda i: (i, 0, 0)),
        ],
        out_specs=pl.BlockSpec((1, _LANES), lambda i: (0, 0)),
        out_shape=jax.ShapeDtypeStruct((1, _LANES), jnp.float32),
        scratch_shapes=[pltpu.VMEM((48, _LANES), jnp.float32)],
    )(bounds, logits, labels_r)

    ece = out[0, 0:1]
    mce = out[0, 1]
    return (ece, mce)
